# baseline (speedup 1.0000x reference)
import functools

import jax
import jax.numpy as jnp
from jax import lax
from jax.experimental import pallas as pl
from jax.experimental.pallas import tpu as pltpu
from jax.experimental.pallas import tpu_sc as plsc

_B, _H, _L1 = 1024, 200, 201
_V, _D, _R = 1000000, 64, 6
_N = _B * _H
_VP = _V // 2
_DP = 2 * _D

_NC, _NS = 2, 16
_NW = _NC * _NS
_RPW = _N // _NW
_CH = 128
_NCH = _RPW // _CH
_NB = 2


def _sc_gather_body(tab_hbm, idx_hbm, out_hbm, idx_v, buf0, buf1,
                    gsem, wsem0, wsem1):
    wid = lax.axis_index("s") * _NC + lax.axis_index("c")
    base = wid * _RPW
    pltpu.sync_copy(idx_hbm.at[pl.ds(base, _RPW)], idx_v)
    bufs = (buf0, buf1)
    wsems = (wsem0, wsem1)

    def superchunk(s, carry):
        for b in range(_NB):
            c = s * _NB + b

            @pl.when(s > 0)
            def _():
                pltpu.make_async_copy(
                    bufs[b], out_hbm.at[pl.ds(base, _CH)], wsems[b]).wait()

            idx_view = idx_v.at[pl.ds(c * _CH, _CH)]
            pltpu.async_copy(tab_hbm.at[idx_view], bufs[b], gsem).wait()
            pltpu.async_copy(
                bufs[b], out_hbm.at[pl.ds(base + c * _CH, _CH)], wsems[b])
        return carry

    lax.fori_loop(0, _NCH // _NB, superchunk, 0)
    for b in range(_NB):
        pltpu.make_async_copy(
            bufs[b], out_hbm.at[pl.ds(base, _CH)], wsems[b]).wait()


@functools.cache
def _make_sc_gather():
    return pl.kernel(
        _sc_gather_body,
        out_type=jax.ShapeDtypeStruct((_N, _DP), jnp.float32),
        mesh=plsc.VectorSubcoreMesh(core_axis_name="c", subcore_axis_name="s",
                                    num_cores=_NC, num_subcores=_NS),
        compiler_params=pltpu.CompilerParams(use_tc_tiling_on_sc=False),
        scratch_types=[
            pltpu.VMEM((_RPW,), jnp.int32),
            pltpu.VMEM((_CH, _DP), jnp.float32),
            pltpu.VMEM((_CH, _DP), jnp.float32),
            pltpu.SemaphoreType.DMA,
            pltpu.SemaphoreType.DMA,
            pltpu.SemaphoreType.DMA,
        ],
    )


_CHI = 2048


def _tc_project_body(tin_ref, w2_ref, tout_ref):
    tout_ref[...] = lax.dot_general(
        tin_ref[...], w2_ref[...], (((0,), (0,)), ((), ())),
        preferred_element_type=jnp.float32)


def _tc_project(t2, w2):
    return pl.pallas_call(
        _tc_project_body,
        grid=(pl.cdiv(_VP, _CHI),),
        in_specs=[pl.BlockSpec((_DP, _CHI), lambda i: (0, i)),
                  pl.BlockSpec((_DP, _DP), lambda i: (0, 0))],
        out_specs=pl.BlockSpec((_CHI, _DP), lambda i: (i, 0)),
        out_shape=jax.ShapeDtypeStruct((_VP, _DP), jnp.float32),
    )(t2, w2)


_RT = 2048


def _tc_post_body(gc_ref, ge_ref, cgap_ref, egap_ref, cpar_ref, epar_ref,
                  erat_ref, cu_ref, cc_ref, eu_ref, ce_ref, rp_ref,
                  co_ref, eo_ref):
    cpick = jnp.where(cpar_ref[...] == 0, gc_ref[:, :_D], gc_ref[:, _D:])
    lgc = jnp.log(cgap_ref[...] + 1.0)
    co_ref[...] = cpick + lgc * cu_ref[...] + cc_ref[...]
    epick = jnp.where(epar_ref[...] == 0, ge_ref[:, :_D], ge_ref[:, _D:])
    lge = jnp.log(egap_ref[...] + 1.0)
    onehot = (erat_ref[...] == lax.broadcasted_iota(
        jnp.int32, (_RT, 8), 1)).astype(jnp.float32)
    eo_ref[...] = (
        epick + lge * eu_ref[...]
        + jnp.dot(onehot, rp_ref[...], preferred_element_type=jnp.float32)
        + ce_ref[...])


def _tc_post(gc, ge, cgap, egap, cpar, epar, erat, cu, cc, eu, ce, rp):
    n_blk = _N // _RT
    wide_spec = pl.BlockSpec((_RT, _DP), lambda i: (i, 0))
    out_spec = pl.BlockSpec((_RT, _D), lambda i: (i, 0))
    col_spec = pl.BlockSpec((_RT, 1), lambda i: (i, 0))

    def small(shape):
        return pl.BlockSpec(shape, lambda i: (0, 0))

    return pl.pallas_call(
        _tc_post_body,
        grid=(n_blk,),
        in_specs=[
            wide_spec, wide_spec, col_spec, col_spec, col_spec, col_spec,
            col_spec,
            small((1, _D)), small((1, _D)), small((1, _D)), small((1, _D)),
            small((8, _D)),
        ],
        out_specs=[out_spec, out_spec],
        out_shape=[
            jax.ShapeDtypeStruct((_N, _D), jnp.float32),
            jax.ShapeDtypeStruct((_N, _D), jnp.float32),
        ],
    )(gc, ge, cgap, egap, cpar, epar, erat, cu, cc, eu, ce, rp)


def kernel(row0, row1, row2, row3, row4, row5, row6, row7, click_table,
           exposure_table, rating_table, ts_w, ts_b, exp_w, exp_b, clk_w,
           clk_b):
    del row2, row3, row7
    item_time = row6[:, -1]
    cgap = (item_time[None, :] - row1.T).reshape(_N, 1)
    egap = (item_time[None, :] - row6[:, :-1].T).reshape(_N, 1)
    erat = row5[:, :-1].T.reshape(_N, 1).astype(jnp.int32)
    cidx = row0.T.reshape(_N).astype(jnp.int32)
    eidx = row4[:, :-1].T.reshape(_N).astype(jnp.int32)
    cpar = (cidx >= _VP).astype(jnp.int32)
    epar = (eidx >= _VP).astype(jnp.int32)
    csidx = cidx - cpar * _VP
    esidx = eidx - epar * _VP

    wc = clk_w[:_D]
    we = exp_w[:_D]
    cu = ts_w @ clk_w[_D:2 * _D]
    cc = (ts_b @ clk_w[_D:2 * _D] + rating_table[2] @ clk_w[2 * _D:]
          + clk_b)[None, :]
    eu = ts_w @ exp_w[_D:2 * _D]
    ce = (ts_b @ exp_w[_D:2 * _D] + exp_b)[None, :]
    rp = jnp.zeros((8, _D), jnp.float32).at[:_R].set(
        rating_table @ exp_w[2 * _D:])

    def interleave(w):
        w2 = jnp.zeros((_DP, _DP), jnp.float32)
        return w2.at[0::2, :_D].set(w).at[1::2, _D:].set(w)

    sc_gather = _make_sc_gather()
    pc = _tc_project(click_table.T.reshape(_DP, _VP), interleave(wc))
    gc = sc_gather(pc, csidx)
    pe = _tc_project(exposure_table.T.reshape(_DP, _VP), interleave(we))
    ge = sc_gather(pe, esidx)
    co, eo = _tc_post(gc, ge, cgap, egap, cpar.reshape(_N, 1),
                      epar.reshape(_N, 1), erat, cu, cc, eu, ce, rp)
    return (co.reshape(_H, _B, _D).swapaxes(0, 1),
            eo.reshape(_H, _B, _D).swapaxes(0, 1))


# speedup vs baseline: 2.3526x; 2.3526x over previous
import functools

import jax
import jax.numpy as jnp
from jax import lax
from jax.experimental import pallas as pl
from jax.experimental.pallas import tpu as pltpu
from jax.experimental.pallas import tpu_sc as plsc

_B, _H, _L1 = 1024, 200, 201
_V, _D, _R = 1000000, 64, 6
_N = _B * _H
_VP = _V // 2
_DP = 2 * _D

_NC, _NS = 2, 16
_NW = _NC * _NS
_RPW = _N // _NW
_CH = 128
_NCH = _RPW // _CH
_NB = 2


def _sc_gather_body(tab_hbm, idx_hbm, out_hbm, idx_v, buf0, buf1,
                    gsem, wsem0, wsem1):
    wid = lax.axis_index("s") * _NC + lax.axis_index("c")
    base = wid * _RPW
    pltpu.sync_copy(idx_hbm.at[pl.ds(base, _RPW)], idx_v)
    bufs = (buf0, buf1)
    wsems = (wsem0, wsem1)

    def superchunk(s, carry):
        for b in range(_NB):
            c = s * _NB + b

            @pl.when(s > 0)
            def _():
                pltpu.make_async_copy(
                    bufs[b], out_hbm.at[pl.ds(base, _CH)], wsems[b]).wait()

            idx_view = idx_v.at[pl.ds(c * _CH, _CH)]
            pltpu.async_copy(tab_hbm.at[idx_view], bufs[b], gsem).wait()
            pltpu.async_copy(
                bufs[b], out_hbm.at[pl.ds(base + c * _CH, _CH)], wsems[b])
        return carry

    lax.fori_loop(0, _NCH // _NB, superchunk, 0)
    for b in range(_NB):
        pltpu.make_async_copy(
            bufs[b], out_hbm.at[pl.ds(base, _CH)], wsems[b]).wait()


@functools.cache
def _make_sc_gather():
    return pl.kernel(
        _sc_gather_body,
        out_type=jax.ShapeDtypeStruct((_N, _DP), jnp.float32),
        mesh=plsc.VectorSubcoreMesh(core_axis_name="c", subcore_axis_name="s",
                                    num_cores=_NC, num_subcores=_NS),
        compiler_params=pltpu.CompilerParams(use_tc_tiling_on_sc=False),
        scratch_types=[
            pltpu.VMEM((_RPW,), jnp.int32),
            pltpu.VMEM((_CH, _DP), jnp.float32),
            pltpu.VMEM((_CH, _DP), jnp.float32),
            pltpu.SemaphoreType.DMA,
            pltpu.SemaphoreType.DMA,
            pltpu.SemaphoreType.DMA,
        ],
    )


_CHI = 2048
_NPB = pl.cdiv(_V, 2 * _CHI)
_VB = _NPB * _CHI
_SPLIT = (_NPB - 1) * _CHI


def _tc_project_body(ta_ref, tb_ref, w_ref, tout_ref):
    ya = lax.dot_general(ta_ref[...], w_ref[...], (((0,), (0,)), ((), ())),
                         preferred_element_type=jnp.float32)
    yb = lax.dot_general(tb_ref[...], w_ref[...], (((0,), (0,)), ((), ())),
                         preferred_element_type=jnp.float32)
    tout_ref[...] = jnp.concatenate([ya, yb], axis=1)


def _tc_project(t_t, w):
    return pl.pallas_call(
        _tc_project_body,
        grid=(_NPB,),
        in_specs=[pl.BlockSpec((_D, _CHI), lambda i: (0, i)),
                  pl.BlockSpec((_D, _CHI), lambda i: (0, i + _NPB - 1)),
                  pl.BlockSpec((_D, _D), lambda i: (0, 0))],
        out_specs=pl.BlockSpec((_CHI, _DP), lambda i: (i, 0)),
        out_shape=jax.ShapeDtypeStruct((_VB, _DP), jnp.float32),
    )(t_t, t_t, w)


_RT = 2048


def _tc_post_body(gc_ref, ge_ref, cgap_ref, egap_ref, cpar_ref, epar_ref,
                  erat_ref, cu_ref, cc_ref, eu_ref, ce_ref, rp_ref,
                  co_ref, eo_ref):
    cpick = jnp.where(cpar_ref[...] == 0, gc_ref[:, :_D], gc_ref[:, _D:])
    lgc = jnp.log(cgap_ref[...] + 1.0)
    co_ref[...] = cpick + lgc * cu_ref[...] + cc_ref[...]
    epick = jnp.where(epar_ref[...] == 0, ge_ref[:, :_D], ge_ref[:, _D:])
    lge = jnp.log(egap_ref[...] + 1.0)
    onehot = (erat_ref[...] == lax.broadcasted_iota(
        jnp.int32, (_RT, 8), 1)).astype(jnp.float32)
    eo_ref[...] = (
        epick + lge * eu_ref[...]
        + jnp.dot(onehot, rp_ref[...], preferred_element_type=jnp.float32)
        + ce_ref[...])


def _tc_post(gc, ge, cgap, egap, cpar, epar, erat, cu, cc, eu, ce, rp):
    n_blk = _N // _RT
    wide_spec = pl.BlockSpec((_RT, _DP), lambda i: (i, 0))
    out_spec = pl.BlockSpec((_RT, _D), lambda i: (i, 0))
    col_spec = pl.BlockSpec((_RT, 1), lambda i: (i, 0))

    def small(shape):
        return pl.BlockSpec(shape, lambda i: (0, 0))

    return pl.pallas_call(
        _tc_post_body,
        grid=(n_blk,),
        in_specs=[
            wide_spec, wide_spec, col_spec, col_spec, col_spec, col_spec,
            col_spec,
            small((1, _D)), small((1, _D)), small((1, _D)), small((1, _D)),
            small((8, _D)),
        ],
        out_specs=[out_spec, out_spec],
        out_shape=[
            jax.ShapeDtypeStruct((_N, _D), jnp.float32),
            jax.ShapeDtypeStruct((_N, _D), jnp.float32),
        ],
    )(gc, ge, cgap, egap, cpar, epar, erat, cu, cc, eu, ce, rp)


def kernel(row0, row1, row2, row3, row4, row5, row6, row7, click_table,
           exposure_table, rating_table, ts_w, ts_b, exp_w, exp_b, clk_w,
           clk_b):
    del row2, row3, row7
    item_time = row6[:, -1]
    cgap = (item_time[None, :] - row1.T).reshape(_N, 1)
    egap = (item_time[None, :] - row6[:, :-1].T).reshape(_N, 1)
    erat = row5[:, :-1].T.reshape(_N, 1).astype(jnp.int32)
    cidx = row0.T.reshape(_N).astype(jnp.int32)
    eidx = row4[:, :-1].T.reshape(_N).astype(jnp.int32)
    cpar = (cidx >= _SPLIT).astype(jnp.int32)
    epar = (eidx >= _SPLIT).astype(jnp.int32)
    csidx = cidx - cpar * _SPLIT
    esidx = eidx - epar * _SPLIT

    wc = clk_w[:_D]
    we = exp_w[:_D]
    cu = ts_w @ clk_w[_D:2 * _D]
    cc = (ts_b @ clk_w[_D:2 * _D] + rating_table[2] @ clk_w[2 * _D:]
          + clk_b)[None, :]
    eu = ts_w @ exp_w[_D:2 * _D]
    ce = (ts_b @ exp_w[_D:2 * _D] + exp_b)[None, :]
    rp = jnp.zeros((8, _D), jnp.float32).at[:_R].set(
        rating_table @ exp_w[2 * _D:])

    sc_gather = _make_sc_gather()
    pc = _tc_project(click_table.T, wc)
    gc = sc_gather(pc, csidx)
    pe = _tc_project(exposure_table.T, we)
    ge = sc_gather(pe, esidx)
    co, eo = _tc_post(gc, ge, cgap, egap, cpar.reshape(_N, 1),
                      epar.reshape(_N, 1), erat, cu, cc, eu, ce, rp)
    return (co.reshape(_H, _B, _D).swapaxes(0, 1),
            eo.reshape(_H, _B, _D).swapaxes(0, 1))


# speedup vs baseline: 2.7251x; 1.1583x over previous
import functools

import jax
import jax.numpy as jnp
from jax import lax
from jax.experimental import pallas as pl
from jax.experimental.pallas import tpu as pltpu
from jax.experimental.pallas import tpu_sc as plsc

_B, _H, _L1 = 1024, 200, 201
_V, _D, _R = 1000000, 64, 6
_N = _B * _H
_VP = _V // 2
_DP = 2 * _D

_NC, _NS = 2, 16
_NW = _NC * _NS
_RPW = _N // _NW
_CH = 128
_NCH = _RPW // _CH
_NB = 2


def _sc_gather_body(tab_hbm, idx_hbm, out_hbm, idx_v, buf0, buf1,
                    gsem, wsem0, wsem1):
    wid = lax.axis_index("s") * _NC + lax.axis_index("c")
    base = wid * _RPW
    pltpu.sync_copy(idx_hbm.at[pl.ds(base, _RPW)], idx_v)
    bufs = (buf0, buf1)
    wsems = (wsem0, wsem1)

    def superchunk(s, carry):
        for b in range(_NB):
            c = s * _NB + b

            @pl.when(s > 0)
            def _():
                pltpu.make_async_copy(
                    bufs[b], out_hbm.at[pl.ds(base, _CH)], wsems[b]).wait()

            idx_view = idx_v.at[pl.ds(c * _CH, _CH)]
            pltpu.async_copy(tab_hbm.at[idx_view], bufs[b], gsem).wait()
            pltpu.async_copy(
                bufs[b], out_hbm.at[pl.ds(base + c * _CH, _CH)], wsems[b])
        return carry

    lax.fori_loop(0, _NCH // _NB, superchunk, 0)
    for b in range(_NB):
        pltpu.make_async_copy(
            bufs[b], out_hbm.at[pl.ds(base, _CH)], wsems[b]).wait()


@functools.cache
def _make_sc_gather():
    return pl.kernel(
        _sc_gather_body,
        out_type=jax.ShapeDtypeStruct((_N, _DP), jnp.float32),
        mesh=plsc.VectorSubcoreMesh(core_axis_name="c", subcore_axis_name="s",
                                    num_cores=_NC, num_subcores=_NS),
        compiler_params=pltpu.CompilerParams(use_tc_tiling_on_sc=False),
        scratch_types=[
            pltpu.VMEM((_RPW,), jnp.int32),
            pltpu.VMEM((_CH, _DP), jnp.float32),
            pltpu.VMEM((_CH, _DP), jnp.float32),
            pltpu.SemaphoreType.DMA,
            pltpu.SemaphoreType.DMA,
            pltpu.SemaphoreType.DMA,
        ],
    )


_CHI = 8192
_NPB = pl.cdiv(_V, 2 * _CHI)
_VB = _NPB * _CHI
_SPLIT = (_NPB - 1) * _CHI


def _tc_project_body(ta_ref, tb_ref, w_ref, tout_ref):
    ya = lax.dot_general(ta_ref[...], w_ref[...], (((0,), (0,)), ((), ())),
                         preferred_element_type=jnp.float32)
    yb = lax.dot_general(tb_ref[...], w_ref[...], (((0,), (0,)), ((), ())),
                         preferred_element_type=jnp.float32)
    tout_ref[...] = jnp.concatenate([ya, yb], axis=1)


def _tc_project(t_t, w):
    return pl.pallas_call(
        _tc_project_body,
        grid=(_NPB,),
        in_specs=[pl.BlockSpec((_D, _CHI), lambda i: (0, i)),
                  pl.BlockSpec((_D, _CHI), lambda i: (0, i + _NPB - 1)),
                  pl.BlockSpec((_D, _D), lambda i: (0, 0))],
        out_specs=pl.BlockSpec((_CHI, _DP), lambda i: (i, 0)),
        out_shape=jax.ShapeDtypeStruct((_VB, _DP), jnp.float32),
    )(t_t, t_t, w)


_RT = 2048


def _tc_post_body(gc_ref, ge_ref, cgap_ref, egap_ref, cpar_ref, epar_ref,
                  erat_ref, cu_ref, cc_ref, eu_ref, ce_ref, rp_ref,
                  co_ref, eo_ref):
    cpick = jnp.where(cpar_ref[...] == 0, gc_ref[:, :_D], gc_ref[:, _D:])
    lgc = jnp.log(cgap_ref[...] + 1.0)
    co_ref[...] = cpick + lgc * cu_ref[...] + cc_ref[...]
    epick = jnp.where(epar_ref[...] == 0, ge_ref[:, :_D], ge_ref[:, _D:])
    lge = jnp.log(egap_ref[...] + 1.0)
    onehot = (erat_ref[...] == lax.broadcasted_iota(
        jnp.int32, (_RT, 8), 1)).astype(jnp.float32)
    eo_ref[...] = (
        epick + lge * eu_ref[...]
        + jnp.dot(onehot, rp_ref[...], preferred_element_type=jnp.float32)
        + ce_ref[...])


def _tc_post(gc, ge, cgap, egap, cpar, epar, erat, cu, cc, eu, ce, rp):
    n_blk = _N // _RT
    wide_spec = pl.BlockSpec((_RT, _DP), lambda i: (i, 0))
    out_spec = pl.BlockSpec((_RT, _D), lambda i: (i, 0))
    col_spec = pl.BlockSpec((_RT, 1), lambda i: (i, 0))

    def small(shape):
        return pl.BlockSpec(shape, lambda i: (0, 0))

    return pl.pallas_call(
        _tc_post_body,
        grid=(n_blk,),
        in_specs=[
            wide_spec, wide_spec, col_spec, col_spec, col_spec, col_spec,
            col_spec,
            small((1, _D)), small((1, _D)), small((1, _D)), small((1, _D)),
            small((8, _D)),
        ],
        out_specs=[out_spec, out_spec],
        out_shape=[
            jax.ShapeDtypeStruct((_N, _D), jnp.float32),
            jax.ShapeDtypeStruct((_N, _D), jnp.float32),
        ],
    )(gc, ge, cgap, egap, cpar, epar, erat, cu, cc, eu, ce, rp)


def kernel(row0, row1, row2, row3, row4, row5, row6, row7, click_table,
           exposure_table, rating_table, ts_w, ts_b, exp_w, exp_b, clk_w,
           clk_b):
    del row2, row3, row7
    item_time = row6[:, -1]
    cgap = (item_time[None, :] - row1.T).reshape(_N, 1)
    egap = (item_time[None, :] - row6[:, :-1].T).reshape(_N, 1)
    erat = row5[:, :-1].T.reshape(_N, 1).astype(jnp.int32)
    cidx = row0.T.reshape(_N).astype(jnp.int32)
    eidx = row4[:, :-1].T.reshape(_N).astype(jnp.int32)
    cpar = (cidx >= _SPLIT).astype(jnp.int32)
    epar = (eidx >= _SPLIT).astype(jnp.int32)
    csidx = cidx - cpar * _SPLIT
    esidx = eidx - epar * _SPLIT

    wc = clk_w[:_D]
    we = exp_w[:_D]
    cu = ts_w @ clk_w[_D:2 * _D]
    cc = (ts_b @ clk_w[_D:2 * _D] + rating_table[2] @ clk_w[2 * _D:]
          + clk_b)[None, :]
    eu = ts_w @ exp_w[_D:2 * _D]
    ce = (ts_b @ exp_w[_D:2 * _D] + exp_b)[None, :]
    rp = jnp.zeros((8, _D), jnp.float32).at[:_R].set(
        rating_table @ exp_w[2 * _D:])

    sc_gather = _make_sc_gather()
    pc = _tc_project(click_table.T, wc)
    gc = sc_gather(pc, csidx)
    pe = _tc_project(exposure_table.T, we)
    ge = sc_gather(pe, esidx)
    co, eo = _tc_post(gc, ge, cgap, egap, cpar.reshape(_N, 1),
                      epar.reshape(_N, 1), erat, cu, cc, eu, ce, rp)
    return (co.reshape(_H, _B, _D).swapaxes(0, 1),
            eo.reshape(_H, _B, _D).swapaxes(0, 1))


# speedup vs baseline: 2.7880x; 1.0231x over previous
import functools

import jax
import jax.numpy as jnp
from jax import lax
from jax.experimental import pallas as pl
from jax.experimental.pallas import tpu as pltpu
from jax.experimental.pallas import tpu_sc as plsc

_B, _H, _L1 = 1024, 200, 201
_V, _D, _R = 1000000, 64, 6
_N = _B * _H
_VP = _V // 2
_DP = 2 * _D

_NC, _NS = 2, 16
_NW = _NC * _NS
_RPW = _N // _NW
_CH = 128
_NCH = _RPW // _CH
_NB = 2


def _sc_gather_body(tab_hbm, idx_hbm, out_hbm, idx_v, buf0, buf1,
                    gsem, wsem0, wsem1):
    wid = lax.axis_index("s") * _NC + lax.axis_index("c")
    base = wid * _RPW
    pltpu.sync_copy(idx_hbm.at[pl.ds(base, _RPW)], idx_v)
    bufs = (buf0, buf1)
    wsems = (wsem0, wsem1)

    def superchunk(s, carry):
        for b in range(_NB):
            c = s * _NB + b

            @pl.when(s > 0)
            def _():
                pltpu.make_async_copy(
                    bufs[b], out_hbm.at[pl.ds(base, _CH)], wsems[b]).wait()

            idx_view = idx_v.at[pl.ds(c * _CH, _CH)]
            pltpu.async_copy(tab_hbm.at[idx_view], bufs[b], gsem).wait()
            pltpu.async_copy(
                bufs[b], out_hbm.at[pl.ds(base + c * _CH, _CH)], wsems[b])
        return carry

    lax.fori_loop(0, _NCH // _NB, superchunk, 0)
    for b in range(_NB):
        pltpu.make_async_copy(
            bufs[b], out_hbm.at[pl.ds(base, _CH)], wsems[b]).wait()


@functools.cache
def _make_sc_gather():
    return pl.kernel(
        _sc_gather_body,
        out_type=jax.ShapeDtypeStruct((_N, _DP), jnp.float32),
        mesh=plsc.VectorSubcoreMesh(core_axis_name="c", subcore_axis_name="s",
                                    num_cores=_NC, num_subcores=_NS),
        compiler_params=pltpu.CompilerParams(use_tc_tiling_on_sc=False),
        scratch_types=[
            pltpu.VMEM((_RPW,), jnp.int32),
            pltpu.VMEM((_CH, _DP), jnp.float32),
            pltpu.VMEM((_CH, _DP), jnp.float32),
            pltpu.SemaphoreType.DMA,
            pltpu.SemaphoreType.DMA,
            pltpu.SemaphoreType.DMA,
        ],
    )


_CHI = 16384
_NPB = pl.cdiv(_V, 2 * _CHI)
_VB = _NPB * _CHI
_SPLIT = (_NPB - 1) * _CHI


def _tc_project_body(ta_ref, tb_ref, w_ref, tout_ref):
    ya = lax.dot_general(ta_ref[...], w_ref[...], (((0,), (0,)), ((), ())),
                         preferred_element_type=jnp.float32)
    yb = lax.dot_general(tb_ref[...], w_ref[...], (((0,), (0,)), ((), ())),
                         preferred_element_type=jnp.float32)
    tout_ref[...] = jnp.concatenate([ya, yb], axis=1)


def _tc_project(t_t, w):
    return pl.pallas_call(
        _tc_project_body,
        grid=(_NPB,),
        in_specs=[pl.BlockSpec((_D, _CHI), lambda i: (0, i)),
                  pl.BlockSpec((_D, _CHI), lambda i: (0, i + _NPB - 1)),
                  pl.BlockSpec((_D, _D), lambda i: (0, 0))],
        out_specs=pl.BlockSpec((_CHI, _DP), lambda i: (i, 0)),
        out_shape=jax.ShapeDtypeStruct((_VB, _DP), jnp.float32),
    )(t_t, t_t, w)


_RT = 2048


def _tc_post_body(gc_ref, ge_ref, cgap_ref, egap_ref, cpar_ref, epar_ref,
                  erat_ref, cu_ref, cc_ref, eu_ref, ce_ref, rp_ref,
                  co_ref, eo_ref):
    cpick = jnp.where(cpar_ref[...] == 0, gc_ref[:, :_D], gc_ref[:, _D:])
    lgc = jnp.log(cgap_ref[...] + 1.0)
    co_ref[...] = cpick + lgc * cu_ref[...] + cc_ref[...]
    epick = jnp.where(epar_ref[...] == 0, ge_ref[:, :_D], ge_ref[:, _D:])
    lge = jnp.log(egap_ref[...] + 1.0)
    onehot = (erat_ref[...] == lax.broadcasted_iota(
        jnp.int32, (_RT, 8), 1)).astype(jnp.float32)
    eo_ref[...] = (
        epick + lge * eu_ref[...]
        + jnp.dot(onehot, rp_ref[...], preferred_element_type=jnp.float32)
        + ce_ref[...])


def _tc_post(gc, ge, cgap, egap, cpar, epar, erat, cu, cc, eu, ce, rp):
    n_blk = _N // _RT
    wide_spec = pl.BlockSpec((_RT, _DP), lambda i: (i, 0))
    out_spec = pl.BlockSpec((_RT, _D), lambda i: (i, 0))
    col_spec = pl.BlockSpec((_RT, 1), lambda i: (i, 0))

    def small(shape):
        return pl.BlockSpec(shape, lambda i: (0, 0))

    return pl.pallas_call(
        _tc_post_body,
        grid=(n_blk,),
        in_specs=[
            wide_spec, wide_spec, col_spec, col_spec, col_spec, col_spec,
            col_spec,
            small((1, _D)), small((1, _D)), small((1, _D)), small((1, _D)),
            small((8, _D)),
        ],
        out_specs=[out_spec, out_spec],
        out_shape=[
            jax.ShapeDtypeStruct((_N, _D), jnp.float32),
            jax.ShapeDtypeStruct((_N, _D), jnp.float32),
        ],
    )(gc, ge, cgap, egap, cpar, epar, erat, cu, cc, eu, ce, rp)


def kernel(row0, row1, row2, row3, row4, row5, row6, row7, click_table,
           exposure_table, rating_table, ts_w, ts_b, exp_w, exp_b, clk_w,
           clk_b):
    del row2, row3, row7
    item_time = row6[:, -1]
    cgap = (item_time[None, :] - row1.T).reshape(_N, 1)
    egap = (item_time[None, :] - row6[:, :-1].T).reshape(_N, 1)
    erat = row5[:, :-1].T.reshape(_N, 1).astype(jnp.int32)
    cidx = row0.T.reshape(_N).astype(jnp.int32)
    eidx = row4[:, :-1].T.reshape(_N).astype(jnp.int32)
    cpar = (cidx >= _SPLIT).astype(jnp.int32)
    epar = (eidx >= _SPLIT).astype(jnp.int32)
    csidx = cidx - cpar * _SPLIT
    esidx = eidx - epar * _SPLIT

    wc = clk_w[:_D]
    we = exp_w[:_D]
    cu = ts_w @ clk_w[_D:2 * _D]
    cc = (ts_b @ clk_w[_D:2 * _D] + rating_table[2] @ clk_w[2 * _D:]
          + clk_b)[None, :]
    eu = ts_w @ exp_w[_D:2 * _D]
    ce = (ts_b @ exp_w[_D:2 * _D] + exp_b)[None, :]
    rp = jnp.zeros((8, _D), jnp.float32).at[:_R].set(
        rating_table @ exp_w[2 * _D:])

    sc_gather = _make_sc_gather()
    pc = _tc_project(click_table.T, wc)
    gc = sc_gather(pc, csidx)
    pe = _tc_project(exposure_table.T, we)
    ge = sc_gather(pe, esidx)
    co, eo = _tc_post(gc, ge, cgap, egap, cpar.reshape(_N, 1),
                      epar.reshape(_N, 1), erat, cu, cc, eu, ce, rp)
    return (co.reshape(_H, _B, _D).swapaxes(0, 1),
            eo.reshape(_H, _B, _D).swapaxes(0, 1))
